# Initial kernel scaffold; baseline (speedup 1.0000x reference)
import functools
import jax
import jax.numpy as jnp
from jax import lax
from jax.experimental import pallas as pl
from jax.experimental.pallas import tpu as pltpu
from jax.experimental.pallas import tpu_sc as plsc

N = 10000
E = 160000
D = 256
HALF = 128
EPS = 1e-5
SLOPE = 0.01

NC = 2
NS = 16

E_PER_T = E // NS
CHUNK = 400
N_CHUNKS = E_PER_T // CHUNK
ROWS_PER_T = N // NS
ZROWS = 125

DCHUNK = 1000
DN_CHUNKS = E_PER_T // DCHUNK
DW = 16

_mesh = plsc.VectorSubcoreMesh(core_axis_name="c", subcore_axis_name="s")


def _leaky(x):
    return jnp.where(x >= 0, x, SLOPE * x)


def _deg_body(src_hbm, dst_hbm, out_hbm, idx_v, ones_v, deg_sh, sem):
    cid = lax.axis_index("c")
    sid = lax.axis_index("s")

    def fill_ones(r, _):
        ones_v[r, :] = jnp.ones((16,), jnp.float32)
        return 0

    def zero_rows(r, _):
        ones_v[r, :] = jnp.zeros((16,), jnp.float32)
        return 0

    lax.fori_loop(0, ZROWS, zero_rows, 0)
    for i in range(ROWS_PER_T // ZROWS):
        pltpu.sync_copy(ones_v.at[pl.ds(0, ZROWS)],
                        deg_sh.at[pl.ds(sid * ROWS_PER_T + i * ZROWS, ZROWS)])
    lax.fori_loop(0, DCHUNK, fill_ones, 0)
    plsc.subcore_barrier()

    base = sid * E_PER_T

    def chunk(k, _):
        off = base + k * DCHUNK

        @pl.when(cid == 0)
        def _():
            pltpu.sync_copy(src_hbm.at[pl.ds(off, DCHUNK)], idx_v)

        @pl.when(cid == 1)
        def _():
            pltpu.sync_copy(dst_hbm.at[pl.ds(off, DCHUNK)], idx_v)

        pltpu.sync_copy(ones_v, deg_sh.at[idx_v], add=True)
        return 0

    lax.fori_loop(0, DN_CHUNKS, chunk, 0)
    plsc.subcore_barrier()
    pltpu.sync_copy(deg_sh.at[pl.ds(sid * ROWS_PER_T, ROWS_PER_T)],
                    out_hbm.at[cid, pl.ds(sid * ROWS_PER_T, ROWS_PER_T)])


_deg_call = pl.kernel(
    _deg_body,
    out_type=jax.ShapeDtypeStruct((NC, N, DW), jnp.float32),
    mesh=_mesh,
    scratch_types=[
        pltpu.VMEM((DCHUNK,), jnp.int32),
        pltpu.VMEM((DCHUNK, DW), jnp.float32),
        pltpu.VMEM_SHARED((N, DW), jnp.float32),
        pltpu.SemaphoreType.DMA,
    ],
)


def _spmm_body(h_hbm, src_hbm, dst_hbm, ew_hbm, out_hbm,
               sidx_v, didx_v, ew_v, rows_v, zero_v, agg_sh, sem):
    cid = lax.axis_index("c")
    sid = lax.axis_index("s")

    def zero_rows(r, _):
        for j in range(HALF // 16):
            zero_v[r, pl.ds(j * 16, 16)] = jnp.zeros((16,), jnp.float32)
        return 0
    lax.fori_loop(0, ZROWS, zero_rows, 0)
    for i in range(ROWS_PER_T // ZROWS):
        pltpu.sync_copy(zero_v,
                        agg_sh.at[pl.ds(sid * ROWS_PER_T + i * ZROWS, ZROWS)])
    plsc.subcore_barrier()

    base = sid * E_PER_T

    def chunk(k, _):
        off = base + k * CHUNK
        pltpu.sync_copy(src_hbm.at[pl.ds(off, CHUNK)], sidx_v)
        pltpu.sync_copy(dst_hbm.at[pl.ds(off, CHUNK)], didx_v)
        pltpu.sync_copy(ew_hbm.at[pl.ds(off, CHUNK)], ew_v)
        pltpu.async_copy(h_hbm.at[cid].at[sidx_v], rows_v, sem).wait()

        def scale(e, _):
            w = plsc.load_gather(ew_v, [jnp.full((16,), e, jnp.int32)])
            for j in range(HALF // 16):
                sl = pl.ds(j * 16, 16)
                rows_v[e, sl] = rows_v[e, sl] * w
            return 0
        lax.fori_loop(0, CHUNK, scale, 0)

        pltpu.sync_copy(rows_v, agg_sh.at[didx_v], add=True)
        return 0

    lax.fori_loop(0, N_CHUNKS, chunk, 0)
    plsc.subcore_barrier()
    pltpu.sync_copy(agg_sh.at[pl.ds(sid * ROWS_PER_T, ROWS_PER_T)],
                    out_hbm.at[cid, pl.ds(sid * ROWS_PER_T, ROWS_PER_T)])


_spmm_call = pl.kernel(
    _spmm_body,
    out_type=jax.ShapeDtypeStruct((NC, N, HALF), jnp.float32),
    mesh=_mesh,
    scratch_types=[
        pltpu.VMEM((CHUNK,), jnp.int32),
        pltpu.VMEM((CHUNK,), jnp.int32),
        pltpu.VMEM((CHUNK,), jnp.float32),
        pltpu.VMEM((CHUNK, HALF), jnp.float32),
        pltpu.VMEM((ZROWS, HALF), jnp.float32),
        pltpu.VMEM_SHARED((N, HALF), jnp.float32),
        pltpu.SemaphoreType.DMA,
    ],
)


BLK = 1000
NBLK = N // BLK


def _norm_from(deg_block):
    return lax.rsqrt(jnp.maximum(deg_block, 1.0))


def _scale_body(x_ref, degs_ref, out_ref):
    ns = _norm_from(degs_ref[0, :, 0:1])
    x = x_ref[...]
    out_ref[0] = x[:, :HALF] * ns
    out_ref[1] = x[:, HALF:] * ns


def _scale_call(features, degs):
    return pl.pallas_call(
        _scale_body,
        grid=(NBLK,),
        in_specs=[
            pl.BlockSpec((BLK, D), lambda i: (i, 0)),
            pl.BlockSpec((1, BLK, DW), lambda i: (0, i, 0)),
        ],
        out_specs=pl.BlockSpec((NC, BLK, HALF), lambda i: (0, i, 0)),
        out_shape=jax.ShapeDtypeStruct((NC, N, HALF), jnp.float32),
    )(features, degs)


def _mm_body(agg_ref, degs_ref, w_ref, z_ref, st_ref):
    i = pl.program_id(0)
    nd = _norm_from(degs_ref[0, :, 0:1])
    a0 = agg_ref[0] * nd
    a1 = agg_ref[1] * nd
    z = (jnp.dot(a0, w_ref[:HALF, :], preferred_element_type=jnp.float32) +
         jnp.dot(a1, w_ref[HALF:, :], preferred_element_type=jnp.float32))
    z_ref[...] = z
    s1 = jnp.sum(z, axis=0, keepdims=True)
    s2 = jnp.sum(z * z, axis=0, keepdims=True)
    st = jnp.concatenate([s1, s2], axis=0)

    @pl.when(i == 0)
    def _():
        st_ref[...] = st

    @pl.when(i > 0)
    def _():
        st_ref[...] = st_ref[...] + st


def _mm_call(agg, degs, w):
    return pl.pallas_call(
        _mm_body,
        grid=(NBLK,),
        in_specs=[
            pl.BlockSpec((NC, BLK, HALF), lambda i: (0, i, 0)),
            pl.BlockSpec((1, BLK, DW), lambda i: (1, i, 0)),
            pl.BlockSpec((D, D), lambda i: (0, 0)),
        ],
        out_specs=[
            pl.BlockSpec((BLK, D), lambda i: (i, 0)),
            pl.BlockSpec((2, D), lambda i: (0, 0)),
        ],
        out_shape=[
            jax.ShapeDtypeStruct((N, D), jnp.float32),
            jax.ShapeDtypeStruct((2, D), jnp.float32),
        ],
    )(agg, degs, w)


def _gnorm(z, st_ref, alpha_ref, gamma_ref, beta_ref):
    alpha = alpha_ref[...]
    m = st_ref[0:1, :] * (1.0 / N)
    var = st_ref[1:2, :] * (1.0 / N) + (alpha * alpha - 2.0 * alpha) * m * m
    inv = lax.rsqrt(var + EPS)
    return _leaky(gamma_ref[...] * inv * (z - alpha * m) + beta_ref[...])


def _gn_scale_body(z_ref, st_ref, degs_ref, a_ref, g_ref, b_ref,
                   out_ref, r_ref):
    i = pl.program_id(0)
    h = _gnorm(z_ref[...], st_ref, a_ref, g_ref, b_ref)
    r = jnp.sum(h, axis=0, keepdims=True)

    @pl.when(i == 0)
    def _():
        r_ref[...] = r

    @pl.when(i > 0)
    def _():
        r_ref[...] = r_ref[...] + r

    ns = _norm_from(degs_ref[0, :, 0:1])
    hs = h * ns
    out_ref[0] = hs[:, :HALF]
    out_ref[1] = hs[:, HALF:]


def _gn_scale_call(z, st, degs, alpha, gamma, beta):
    return pl.pallas_call(
        _gn_scale_body,
        grid=(NBLK,),
        in_specs=[
            pl.BlockSpec((BLK, D), lambda i: (i, 0)),
            pl.BlockSpec((2, D), lambda i: (0, 0)),
            pl.BlockSpec((1, BLK, DW), lambda i: (0, i, 0)),
            pl.BlockSpec((1, D), lambda i: (0, 0)),
            pl.BlockSpec((1, D), lambda i: (0, 0)),
            pl.BlockSpec((1, D), lambda i: (0, 0)),
        ],
        out_specs=[
            pl.BlockSpec((NC, BLK, HALF), lambda i: (0, i, 0)),
            pl.BlockSpec((1, D), lambda i: (0, 0)),
        ],
        out_shape=[
            jax.ShapeDtypeStruct((NC, N, HALF), jnp.float32),
            jax.ShapeDtypeStruct((1, D), jnp.float32),
        ],
    )(z, st, degs, alpha, gamma, beta)


def _final_body(z_ref, st_ref, a_ref, g_ref, b_ref, r1_ref, out_ref, racc):
    i = pl.program_id(0)
    h = _gnorm(z_ref[...], st_ref, a_ref, g_ref, b_ref)
    r = jnp.sum(h, axis=0, keepdims=True)

    @pl.when(i == 0)
    def _():
        racc[...] = r

    @pl.when(i > 0)
    def _():
        racc[...] = racc[...] + r

    @pl.when(i == NBLK - 1)
    def _():
        out_ref[0:1, :D] = _leaky(r1_ref[...] * (1.0 / N))
        out_ref[0:1, D:] = _leaky(racc[...] * (1.0 / N))


def _final_call(z, st, alpha, gamma, beta, r1):
    return pl.pallas_call(
        _final_body,
        grid=(NBLK,),
        in_specs=[
            pl.BlockSpec((BLK, D), lambda i: (i, 0)),
            pl.BlockSpec((2, D), lambda i: (0, 0)),
            pl.BlockSpec((1, D), lambda i: (0, 0)),
            pl.BlockSpec((1, D), lambda i: (0, 0)),
            pl.BlockSpec((1, D), lambda i: (0, 0)),
            pl.BlockSpec((1, D), lambda i: (0, 0)),
        ],
        out_specs=pl.BlockSpec((1, 2 * D), lambda i: (0, 0)),
        out_shape=jax.ShapeDtypeStruct((1, 2 * D), jnp.float32),
        scratch_shapes=[pltpu.VMEM((1, D), jnp.float32)],
    )(z, st, alpha, gamma, beta, r1)


@jax.jit
def _run(features, src, dst, edge_weights, W1, W2,
         gn1_alpha, gn1_gamma, gn1_beta, gn2_alpha, gn2_gamma, gn2_beta):
    degs = _deg_call(src, dst)
    h0 = _scale_call(features, degs)
    agg1 = _spmm_call(h0, src, dst, edge_weights)
    z1, st1 = _mm_call(agg1, degs, W1)
    h1s, r1 = _gn_scale_call(z1, st1, degs, gn1_alpha, gn1_gamma, gn1_beta)
    agg2 = _spmm_call(h1s, src, dst, edge_weights)
    z2, st2 = _mm_call(agg2, degs, W2)
    return _final_call(z2, st2, gn2_alpha, gn2_gamma, gn2_beta, r1)


def kernel(features, edge_index, edge_weights, W1, W2,
           gn1_alpha, gn1_gamma, gn1_beta,
           gn2_alpha, gn2_gamma, gn2_beta):
    edge_index = edge_index.astype(jnp.int32)
    src = edge_index[0]
    dst = edge_index[1]
    return _run(features, src, dst, edge_weights, W1, W2,
                gn1_alpha.reshape(1, D), gn1_gamma.reshape(1, D),
                gn1_beta.reshape(1, D), gn2_alpha.reshape(1, D),
                gn2_gamma.reshape(1, D), gn2_beta.reshape(1, D))


# speedup vs baseline: 2.2213x; 2.2213x over previous
import functools
import jax
import jax.numpy as jnp
from jax import lax
from jax.experimental import pallas as pl
from jax.experimental.pallas import tpu as pltpu
from jax.experimental.pallas import tpu_sc as plsc

N = 10000
E = 160000
D = 256
HALF = 128
EPS = 1e-5
SLOPE = 0.01

NC = 2
NS = 16

E_PER_T = E // NS
CHUNK = 80
N_CHUNKS = E_PER_T // CHUNK
CP_ROWS = 624
TAIL = N - NS * CP_ROWS
ZR = 48

DCHUNK = CHUNK
DN_CHUNKS = E_PER_T // DCHUNK
DW = 16

_mesh = plsc.VectorSubcoreMesh(core_axis_name="c", subcore_axis_name="s")


def _leaky(x):
    return jnp.where(x >= 0, x, SLOPE * x)


def _deg_body(src_hbm, dst_hbm, out_hbm, idx_v, ones_v, deg_sh, sem):
    cid = lax.axis_index("c")
    sid = lax.axis_index("s")

    def zero_rows(r, _):
        for j in range(HALF // 16):
            ones_v[r, pl.ds(j * 16, 16)] = jnp.zeros((16,), jnp.float32)
        return 0

    def fill_ones(r, _):
        for j in range(HALF // 16):
            ones_v[r, pl.ds(j * 16, 16)] = jnp.ones((16,), jnp.float32)
        return 0

    lax.fori_loop(0, ZR, zero_rows, 0)
    for i in range(CP_ROWS // ZR):
        pltpu.sync_copy(ones_v.at[pl.ds(0, ZR)],
                        deg_sh.at[pl.ds(sid * CP_ROWS + i * ZR, ZR)])

    @pl.when(sid == 0)
    def _():
        pltpu.sync_copy(ones_v.at[pl.ds(0, TAIL)],
                        deg_sh.at[pl.ds(NS * CP_ROWS, TAIL)])

    lax.fori_loop(0, DCHUNK, fill_ones, 0)
    plsc.subcore_barrier()

    base = sid * E_PER_T

    def chunk(k, _):
        off = base + k * DCHUNK

        @pl.when(cid == 0)
        def _():
            pltpu.sync_copy(src_hbm.at[pl.ds(off, DCHUNK)], idx_v)

        @pl.when(cid == 1)
        def _():
            pltpu.sync_copy(dst_hbm.at[pl.ds(off, DCHUNK)], idx_v)

        pltpu.sync_copy(ones_v, deg_sh.at[idx_v], add=True)
        return 0

    lax.fori_loop(0, DN_CHUNKS, chunk, 0)
    plsc.subcore_barrier()
    pltpu.sync_copy(deg_sh.at[pl.ds(sid * CP_ROWS, CP_ROWS)],
                    out_hbm.at[cid, pl.ds(sid * CP_ROWS, CP_ROWS)])

    @pl.when(sid == 0)
    def _():
        pltpu.sync_copy(deg_sh.at[pl.ds(NS * CP_ROWS, TAIL)],
                        out_hbm.at[cid, pl.ds(NS * CP_ROWS, TAIL)])


_deg_call = pl.kernel(
    _deg_body,
    out_type=jax.ShapeDtypeStruct((NC, N, HALF), jnp.float32),
    mesh=_mesh,
    scratch_types=[
        pltpu.VMEM((DCHUNK,), jnp.int32),
        pltpu.VMEM((DCHUNK, HALF), jnp.float32),
        pltpu.VMEM_SHARED((N, HALF), jnp.float32),
        pltpu.SemaphoreType.DMA,
    ],
)


def _spmm_body(h_hbm, src_hbm, dst_hbm, ew_hbm, out_hbm,
               sidx_v, didx_v, ew_v, rows_v, agg_sh, sem):
    cid = lax.axis_index("c")
    sid = lax.axis_index("s")

    def zero_rows(r, _):
        for j in range(HALF // 16):
            rows_v[r, pl.ds(j * 16, 16)] = jnp.zeros((16,), jnp.float32)
        return 0
    lax.fori_loop(0, ZR, zero_rows, 0)
    for i in range(CP_ROWS // ZR):
        pltpu.sync_copy(rows_v.at[pl.ds(0, ZR)],
                        agg_sh.at[pl.ds(sid * CP_ROWS + i * ZR, ZR)])

    @pl.when(sid == 0)
    def _():
        pltpu.sync_copy(rows_v.at[pl.ds(0, TAIL)],
                        agg_sh.at[pl.ds(NS * CP_ROWS, TAIL)])

    plsc.subcore_barrier()

    base = sid * E_PER_T

    def chunk(k, _):
        off = base + k * CHUNK
        pltpu.sync_copy(src_hbm.at[pl.ds(off, CHUNK)], sidx_v)
        pltpu.sync_copy(dst_hbm.at[pl.ds(off, CHUNK)], didx_v)
        pltpu.sync_copy(ew_hbm.at[pl.ds(off, CHUNK)], ew_v)
        pltpu.async_copy(h_hbm.at[cid].at[sidx_v], rows_v, sem).wait()

        def scale(e, _):
            w = ew_v[e, :]
            for j in range(HALF // 16):
                sl = pl.ds(j * 16, 16)
                rows_v[e, sl] = rows_v[e, sl] * w
            return 0
        lax.fori_loop(0, CHUNK, scale, 0)

        pltpu.sync_copy(rows_v, agg_sh.at[didx_v], add=True)
        return 0

    lax.fori_loop(0, N_CHUNKS, chunk, 0)
    plsc.subcore_barrier()
    pltpu.sync_copy(agg_sh.at[pl.ds(sid * CP_ROWS, CP_ROWS)],
                    out_hbm.at[cid, pl.ds(sid * CP_ROWS, CP_ROWS)])

    @pl.when(sid == 0)
    def _():
        pltpu.sync_copy(agg_sh.at[pl.ds(NS * CP_ROWS, TAIL)],
                        out_hbm.at[cid, pl.ds(NS * CP_ROWS, TAIL)])


_spmm_call = pl.kernel(
    _spmm_body,
    out_type=jax.ShapeDtypeStruct((NC, N, HALF), jnp.float32),
    mesh=_mesh,
    scratch_types=[
        pltpu.VMEM((CHUNK,), jnp.int32),
        pltpu.VMEM((CHUNK,), jnp.int32),
        pltpu.VMEM((CHUNK, 16), jnp.float32),
        pltpu.VMEM((CHUNK, HALF), jnp.float32),
        pltpu.VMEM_SHARED((N, HALF), jnp.float32),
        pltpu.SemaphoreType.DMA,
    ],
)


BLK = 1000
NBLK = N // BLK


def _norm_from(deg_block):
    return lax.rsqrt(jnp.maximum(deg_block, 1.0))


def _scale_body(x_ref, degs_ref, out_ref):
    ns = _norm_from(degs_ref[0, :, 0:1])
    x = x_ref[...]
    out_ref[0] = x[:, :HALF] * ns
    out_ref[1] = x[:, HALF:] * ns


def _scale_call(features, degs):
    return pl.pallas_call(
        _scale_body,
        grid=(NBLK,),
        in_specs=[
            pl.BlockSpec((BLK, D), lambda i: (i, 0)),
            pl.BlockSpec((1, BLK, DW), lambda i: (0, i, 0)),
        ],
        out_specs=pl.BlockSpec((NC, BLK, HALF), lambda i: (0, i, 0)),
        out_shape=jax.ShapeDtypeStruct((NC, N, HALF), jnp.float32),
    )(features, degs)


def _mm_body(agg_ref, degs_ref, w_ref, z_ref, st_ref):
    i = pl.program_id(0)
    nd = _norm_from(degs_ref[0, :, 0:1])
    a0 = agg_ref[0] * nd
    a1 = agg_ref[1] * nd
    z = (jnp.dot(a0, w_ref[:HALF, :], preferred_element_type=jnp.float32) +
         jnp.dot(a1, w_ref[HALF:, :], preferred_element_type=jnp.float32))
    z_ref[...] = z
    s1 = jnp.sum(z, axis=0, keepdims=True)
    s2 = jnp.sum(z * z, axis=0, keepdims=True)
    st = jnp.concatenate([s1, s2], axis=0)

    @pl.when(i == 0)
    def _():
        st_ref[...] = st

    @pl.when(i > 0)
    def _():
        st_ref[...] = st_ref[...] + st


def _mm_call(agg, degs, w):
    return pl.pallas_call(
        _mm_body,
        grid=(NBLK,),
        in_specs=[
            pl.BlockSpec((NC, BLK, HALF), lambda i: (0, i, 0)),
            pl.BlockSpec((1, BLK, DW), lambda i: (1, i, 0)),
            pl.BlockSpec((D, D), lambda i: (0, 0)),
        ],
        out_specs=[
            pl.BlockSpec((BLK, D), lambda i: (i, 0)),
            pl.BlockSpec((2, D), lambda i: (0, 0)),
        ],
        out_shape=[
            jax.ShapeDtypeStruct((N, D), jnp.float32),
            jax.ShapeDtypeStruct((2, D), jnp.float32),
        ],
    )(agg, degs, w)


def _gnorm(z, st_ref, alpha_ref, gamma_ref, beta_ref):
    alpha = alpha_ref[...]
    m = st_ref[0:1, :] * (1.0 / N)
    var = st_ref[1:2, :] * (1.0 / N) + (alpha * alpha - 2.0 * alpha) * m * m
    inv = lax.rsqrt(var + EPS)
    return _leaky(gamma_ref[...] * inv * (z - alpha * m) + beta_ref[...])


def _gn_scale_body(z_ref, st_ref, degs_ref, a_ref, g_ref, b_ref,
                   out_ref, r_ref):
    i = pl.program_id(0)
    h = _gnorm(z_ref[...], st_ref, a_ref, g_ref, b_ref)
    r = jnp.sum(h, axis=0, keepdims=True)

    @pl.when(i == 0)
    def _():
        r_ref[...] = r

    @pl.when(i > 0)
    def _():
        r_ref[...] = r_ref[...] + r

    ns = _norm_from(degs_ref[0, :, 0:1])
    hs = h * ns
    out_ref[0] = hs[:, :HALF]
    out_ref[1] = hs[:, HALF:]


def _gn_scale_call(z, st, degs, alpha, gamma, beta):
    return pl.pallas_call(
        _gn_scale_body,
        grid=(NBLK,),
        in_specs=[
            pl.BlockSpec((BLK, D), lambda i: (i, 0)),
            pl.BlockSpec((2, D), lambda i: (0, 0)),
            pl.BlockSpec((1, BLK, DW), lambda i: (0, i, 0)),
            pl.BlockSpec((1, D), lambda i: (0, 0)),
            pl.BlockSpec((1, D), lambda i: (0, 0)),
            pl.BlockSpec((1, D), lambda i: (0, 0)),
        ],
        out_specs=[
            pl.BlockSpec((NC, BLK, HALF), lambda i: (0, i, 0)),
            pl.BlockSpec((1, D), lambda i: (0, 0)),
        ],
        out_shape=[
            jax.ShapeDtypeStruct((NC, N, HALF), jnp.float32),
            jax.ShapeDtypeStruct((1, D), jnp.float32),
        ],
    )(z, st, degs, alpha, gamma, beta)


def _final_body(z_ref, st_ref, a_ref, g_ref, b_ref, r1_ref, out_ref, racc):
    i = pl.program_id(0)
    h = _gnorm(z_ref[...], st_ref, a_ref, g_ref, b_ref)
    r = jnp.sum(h, axis=0, keepdims=True)

    @pl.when(i == 0)
    def _():
        racc[...] = r

    @pl.when(i > 0)
    def _():
        racc[...] = racc[...] + r

    @pl.when(i == NBLK - 1)
    def _():
        out_ref[0:1, :D] = _leaky(r1_ref[...] * (1.0 / N))
        out_ref[0:1, D:] = _leaky(racc[...] * (1.0 / N))


def _final_call(z, st, alpha, gamma, beta, r1):
    return pl.pallas_call(
        _final_body,
        grid=(NBLK,),
        in_specs=[
            pl.BlockSpec((BLK, D), lambda i: (i, 0)),
            pl.BlockSpec((2, D), lambda i: (0, 0)),
            pl.BlockSpec((1, D), lambda i: (0, 0)),
            pl.BlockSpec((1, D), lambda i: (0, 0)),
            pl.BlockSpec((1, D), lambda i: (0, 0)),
            pl.BlockSpec((1, D), lambda i: (0, 0)),
        ],
        out_specs=pl.BlockSpec((1, 2 * D), lambda i: (0, 0)),
        out_shape=jax.ShapeDtypeStruct((1, 2 * D), jnp.float32),
        scratch_shapes=[pltpu.VMEM((1, D), jnp.float32)],
    )(z, st, alpha, gamma, beta, r1)


@jax.jit
def _run(features, src, dst, edge_weights, W1, W2,
         gn1_alpha, gn1_gamma, gn1_beta, gn2_alpha, gn2_gamma, gn2_beta):
    ew16 = jnp.broadcast_to(edge_weights[:, None], (E, 16))
    degs = _deg_call(src, dst)[:, :, :DW]
    h0 = _scale_call(features, degs)
    agg1 = _spmm_call(h0, src, dst, ew16)
    z1, st1 = _mm_call(agg1, degs, W1)
    h1s, r1 = _gn_scale_call(z1, st1, degs, gn1_alpha, gn1_gamma, gn1_beta)
    agg2 = _spmm_call(h1s, src, dst, ew16)
    z2, st2 = _mm_call(agg2, degs, W2)
    return _final_call(z2, st2, gn2_alpha, gn2_gamma, gn2_beta, r1)


def kernel(features, edge_index, edge_weights, W1, W2,
           gn1_alpha, gn1_gamma, gn1_beta,
           gn2_alpha, gn2_gamma, gn2_beta):
    edge_index = edge_index.astype(jnp.int32)
    src = edge_index[0]
    dst = edge_index[1]
    return _run(features, src, dst, edge_weights, W1, W2,
                gn1_alpha.reshape(1, D), gn1_gamma.reshape(1, D),
                gn1_beta.reshape(1, D), gn2_alpha.reshape(1, D),
                gn2_gamma.reshape(1, D), gn2_beta.reshape(1, D))


# speedup vs baseline: 3.8198x; 1.7196x over previous
import functools
import jax
import jax.numpy as jnp
from jax import lax
from jax.experimental import pallas as pl
from jax.experimental.pallas import tpu as pltpu
from jax.experimental.pallas import tpu_sc as plsc

N = 10000
E = 160000
D = 256
HALF = 128
EPS = 1e-5
SLOPE = 0.01

NC = 2
NS = 16

E_PER_T = E // NS
CHUNK = 40
N_CHUNKS = E_PER_T // CHUNK
CP_ROWS = 624
TAIL = N - NS * CP_ROWS
ZR = 24

DCHUNK = CHUNK
DN_CHUNKS = E_PER_T // DCHUNK
DW = 16

_mesh = plsc.VectorSubcoreMesh(core_axis_name="c", subcore_axis_name="s")


def _leaky(x):
    return jnp.where(x >= 0, x, SLOPE * x)


def _deg_body(src_hbm, dst_hbm, out_hbm, idx_v, ones_v, deg_sh, sem):
    cid = lax.axis_index("c")
    sid = lax.axis_index("s")

    def zero_rows(r, _):
        for j in range(HALF // 16):
            ones_v[r, pl.ds(j * 16, 16)] = jnp.zeros((16,), jnp.float32)
        return 0

    def fill_ones(r, _):
        for j in range(HALF // 16):
            ones_v[r, pl.ds(j * 16, 16)] = jnp.ones((16,), jnp.float32)
        return 0

    lax.fori_loop(0, ZR, zero_rows, 0)
    for i in range(CP_ROWS // ZR):
        pltpu.sync_copy(ones_v.at[pl.ds(0, ZR)],
                        deg_sh.at[pl.ds(sid * CP_ROWS + i * ZR, ZR)])

    @pl.when(sid == 0)
    def _():
        pltpu.sync_copy(ones_v.at[pl.ds(0, TAIL)],
                        deg_sh.at[pl.ds(NS * CP_ROWS, TAIL)])

    lax.fori_loop(0, DCHUNK, fill_ones, 0)
    plsc.subcore_barrier()

    base = sid * E_PER_T

    def chunk(k, _):
        off = base + k * DCHUNK

        @pl.when(cid == 0)
        def _():
            pltpu.sync_copy(src_hbm.at[pl.ds(off, DCHUNK)], idx_v)

        @pl.when(cid == 1)
        def _():
            pltpu.sync_copy(dst_hbm.at[pl.ds(off, DCHUNK)], idx_v)

        pltpu.sync_copy(ones_v, deg_sh.at[idx_v], add=True)
        return 0

    lax.fori_loop(0, DN_CHUNKS, chunk, 0)
    plsc.subcore_barrier()
    pltpu.sync_copy(deg_sh.at[pl.ds(sid * CP_ROWS, CP_ROWS)],
                    out_hbm.at[cid, pl.ds(sid * CP_ROWS, CP_ROWS)])

    @pl.when(sid == 0)
    def _():
        pltpu.sync_copy(deg_sh.at[pl.ds(NS * CP_ROWS, TAIL)],
                        out_hbm.at[cid, pl.ds(NS * CP_ROWS, TAIL)])


_deg_call = pl.kernel(
    _deg_body,
    out_type=jax.ShapeDtypeStruct((NC, N, HALF), jnp.float32),
    mesh=_mesh,
    scratch_types=[
        pltpu.VMEM((DCHUNK,), jnp.int32),
        pltpu.VMEM((DCHUNK, HALF), jnp.float32),
        pltpu.VMEM_SHARED((N, HALF), jnp.float32),
        pltpu.SemaphoreType.DMA,
    ],
)


def _spmm_body(h_hbm, src_hbm, dst_hbm, ew_hbm, out_hbm,
               sidx_v, didx_v, g0_v, g1_v, s0_v, s1_v, e0_v, e1_v, agg_sh,
               sem_g0, sem_g1, sem_e0, sem_e1, sem_s0, sem_s1):
    cid = lax.axis_index("c")
    sid = lax.axis_index("s")
    gbuf = (g0_v, g1_v)
    sbuf = (s0_v, s1_v)
    ebuf = (e0_v, e1_v)
    sem_g = (sem_g0, sem_g1)
    sem_e = (sem_e0, sem_e1)
    sem_s = (sem_s0, sem_s1)
    base = sid * E_PER_T

    def zero_rows(r, _):
        for j in range(HALF // 16):
            g0_v[r, pl.ds(j * 16, 16)] = jnp.zeros((16,), jnp.float32)
        return 0
    lax.fori_loop(0, ZR, zero_rows, 0)
    for i in range(CP_ROWS // ZR):
        pltpu.sync_copy(g0_v.at[pl.ds(0, ZR)],
                        agg_sh.at[pl.ds(sid * CP_ROWS + i * ZR, ZR)])

    @pl.when(sid == 0)
    def _():
        pltpu.sync_copy(g0_v.at[pl.ds(0, TAIL)],
                        agg_sh.at[pl.ds(NS * CP_ROWS, TAIL)])

    pltpu.sync_copy(src_hbm.at[pl.ds(base, E_PER_T)], sidx_v)
    pltpu.sync_copy(dst_hbm.at[pl.ds(base, E_PER_T)], didx_v)

    def _start_gather(b, k):
        pltpu.async_copy(
            h_hbm.at[cid].at[sidx_v.at[pl.ds(k * CHUNK, CHUNK)]],
            gbuf[b], sem_g[b])
        pltpu.async_copy(ew_hbm.at[pl.ds(base + k * CHUNK, CHUNK)],
                         ebuf[b], sem_e[b])

    def _wait_gather(b, k):
        pltpu.make_async_copy(
            h_hbm.at[cid].at[sidx_v.at[pl.ds(k * CHUNK, CHUNK)]],
            gbuf[b], sem_g[b]).wait()
        pltpu.make_async_copy(ew_hbm.at[pl.ds(base + k * CHUNK, CHUNK)],
                              ebuf[b], sem_e[b]).wait()

    def _start_scatter(b, k):
        pltpu.async_copy(
            sbuf[b], agg_sh.at[didx_v.at[pl.ds(k * CHUNK, CHUNK)]],
            sem_s[b], add=True)

    def _wait_scatter(b, k):
        pltpu.make_async_copy(
            sbuf[b], agg_sh.at[didx_v.at[pl.ds(k * CHUNK, CHUNK)]],
            sem_s[b]).wait()

    _start_gather(0, 0)
    _start_gather(1, 1)
    plsc.subcore_barrier()

    def step(k, b):
        _wait_gather(b, k)

        @pl.when(k >= 2)
        def _():
            _wait_scatter(b, k - 2)

        def scale(e, _):
            w = ebuf[b][e, :]
            for j in range(HALF // 16):
                sl = pl.ds(j * 16, 16)
                sbuf[b][e, sl] = gbuf[b][e, sl] * w
            return 0
        lax.fori_loop(0, CHUNK, scale, 0)

        @pl.when(k + 2 < N_CHUNKS)
        def _():
            _start_gather(b, k + 2)

        _start_scatter(b, k)

    def chunk(k, _):
        @pl.when(k % 2 == 0)
        def _():
            step(k, 0)

        @pl.when(k % 2 == 1)
        def _():
            step(k, 1)
        return 0

    lax.fori_loop(0, N_CHUNKS, chunk, 0)
    _wait_scatter((N_CHUNKS - 2) % 2, N_CHUNKS - 2)
    _wait_scatter((N_CHUNKS - 1) % 2, N_CHUNKS - 1)
    plsc.subcore_barrier()
    pltpu.sync_copy(agg_sh.at[pl.ds(sid * CP_ROWS, CP_ROWS)],
                    out_hbm.at[cid, pl.ds(sid * CP_ROWS, CP_ROWS)])

    @pl.when(sid == 0)
    def _():
        pltpu.sync_copy(agg_sh.at[pl.ds(NS * CP_ROWS, TAIL)],
                        out_hbm.at[cid, pl.ds(NS * CP_ROWS, TAIL)])


_spmm_call = pl.kernel(
    _spmm_body,
    out_type=jax.ShapeDtypeStruct((NC, N, HALF), jnp.float32),
    mesh=_mesh,
    scratch_types=[
        pltpu.VMEM((E_PER_T,), jnp.int32),
        pltpu.VMEM((E_PER_T,), jnp.int32),
        pltpu.VMEM((CHUNK, HALF), jnp.float32),
        pltpu.VMEM((CHUNK, HALF), jnp.float32),
        pltpu.VMEM((CHUNK, HALF), jnp.float32),
        pltpu.VMEM((CHUNK, HALF), jnp.float32),
        pltpu.VMEM((CHUNK, 16), jnp.float32),
        pltpu.VMEM((CHUNK, 16), jnp.float32),
        pltpu.VMEM_SHARED((N, HALF), jnp.float32),
        pltpu.SemaphoreType.DMA,
        pltpu.SemaphoreType.DMA,
        pltpu.SemaphoreType.DMA,
        pltpu.SemaphoreType.DMA,
        pltpu.SemaphoreType.DMA,
        pltpu.SemaphoreType.DMA,
    ],
)


BLK = 1000
NBLK = N // BLK


def _norm_from(deg_block):
    return lax.rsqrt(jnp.maximum(deg_block, 1.0))


def _scale_body(x_ref, degs_ref, out_ref):
    ns = _norm_from(degs_ref[0, :, 0:1])
    x = x_ref[...]
    out_ref[0] = x[:, :HALF] * ns
    out_ref[1] = x[:, HALF:] * ns


def _scale_call(features, degs):
    return pl.pallas_call(
        _scale_body,
        grid=(NBLK,),
        in_specs=[
            pl.BlockSpec((BLK, D), lambda i: (i, 0)),
            pl.BlockSpec((1, BLK, DW), lambda i: (0, i, 0)),
        ],
        out_specs=pl.BlockSpec((NC, BLK, HALF), lambda i: (0, i, 0)),
        out_shape=jax.ShapeDtypeStruct((NC, N, HALF), jnp.float32),
    )(features, degs)


def _mm_body(agg_ref, degs_ref, w_ref, z_ref, st_ref):
    i = pl.program_id(0)
    nd = _norm_from(degs_ref[0, :, 0:1])
    a0 = agg_ref[0] * nd
    a1 = agg_ref[1] * nd
    z = (jnp.dot(a0, w_ref[:HALF, :], preferred_element_type=jnp.float32) +
         jnp.dot(a1, w_ref[HALF:, :], preferred_element_type=jnp.float32))
    z_ref[...] = z
    s1 = jnp.sum(z, axis=0, keepdims=True)
    s2 = jnp.sum(z * z, axis=0, keepdims=True)
    st = jnp.concatenate([s1, s2], axis=0)

    @pl.when(i == 0)
    def _():
        st_ref[...] = st

    @pl.when(i > 0)
    def _():
        st_ref[...] = st_ref[...] + st


def _mm_call(agg, degs, w):
    return pl.pallas_call(
        _mm_body,
        grid=(NBLK,),
        in_specs=[
            pl.BlockSpec((NC, BLK, HALF), lambda i: (0, i, 0)),
            pl.BlockSpec((1, BLK, DW), lambda i: (1, i, 0)),
            pl.BlockSpec((D, D), lambda i: (0, 0)),
        ],
        out_specs=[
            pl.BlockSpec((BLK, D), lambda i: (i, 0)),
            pl.BlockSpec((2, D), lambda i: (0, 0)),
        ],
        out_shape=[
            jax.ShapeDtypeStruct((N, D), jnp.float32),
            jax.ShapeDtypeStruct((2, D), jnp.float32),
        ],
    )(agg, degs, w)


def _gnorm(z, st_ref, alpha_ref, gamma_ref, beta_ref):
    alpha = alpha_ref[...]
    m = st_ref[0:1, :] * (1.0 / N)
    var = st_ref[1:2, :] * (1.0 / N) + (alpha * alpha - 2.0 * alpha) * m * m
    inv = lax.rsqrt(var + EPS)
    return _leaky(gamma_ref[...] * inv * (z - alpha * m) + beta_ref[...])


def _gn_scale_body(z_ref, st_ref, degs_ref, a_ref, g_ref, b_ref,
                   out_ref, r_ref):
    i = pl.program_id(0)
    h = _gnorm(z_ref[...], st_ref, a_ref, g_ref, b_ref)
    r = jnp.sum(h, axis=0, keepdims=True)

    @pl.when(i == 0)
    def _():
        r_ref[...] = r

    @pl.when(i > 0)
    def _():
        r_ref[...] = r_ref[...] + r

    ns = _norm_from(degs_ref[0, :, 0:1])
    hs = h * ns
    out_ref[0] = hs[:, :HALF]
    out_ref[1] = hs[:, HALF:]


def _gn_scale_call(z, st, degs, alpha, gamma, beta):
    return pl.pallas_call(
        _gn_scale_body,
        grid=(NBLK,),
        in_specs=[
            pl.BlockSpec((BLK, D), lambda i: (i, 0)),
            pl.BlockSpec((2, D), lambda i: (0, 0)),
            pl.BlockSpec((1, BLK, DW), lambda i: (0, i, 0)),
            pl.BlockSpec((1, D), lambda i: (0, 0)),
            pl.BlockSpec((1, D), lambda i: (0, 0)),
            pl.BlockSpec((1, D), lambda i: (0, 0)),
        ],
        out_specs=[
            pl.BlockSpec((NC, BLK, HALF), lambda i: (0, i, 0)),
            pl.BlockSpec((1, D), lambda i: (0, 0)),
        ],
        out_shape=[
            jax.ShapeDtypeStruct((NC, N, HALF), jnp.float32),
            jax.ShapeDtypeStruct((1, D), jnp.float32),
        ],
    )(z, st, degs, alpha, gamma, beta)


def _final_body(z_ref, st_ref, a_ref, g_ref, b_ref, r1_ref, out_ref, racc):
    i = pl.program_id(0)
    h = _gnorm(z_ref[...], st_ref, a_ref, g_ref, b_ref)
    r = jnp.sum(h, axis=0, keepdims=True)

    @pl.when(i == 0)
    def _():
        racc[...] = r

    @pl.when(i > 0)
    def _():
        racc[...] = racc[...] + r

    @pl.when(i == NBLK - 1)
    def _():
        out_ref[0:1, :D] = _leaky(r1_ref[...] * (1.0 / N))
        out_ref[0:1, D:] = _leaky(racc[...] * (1.0 / N))


def _final_call(z, st, alpha, gamma, beta, r1):
    return pl.pallas_call(
        _final_body,
        grid=(NBLK,),
        in_specs=[
            pl.BlockSpec((BLK, D), lambda i: (i, 0)),
            pl.BlockSpec((2, D), lambda i: (0, 0)),
            pl.BlockSpec((1, D), lambda i: (0, 0)),
            pl.BlockSpec((1, D), lambda i: (0, 0)),
            pl.BlockSpec((1, D), lambda i: (0, 0)),
            pl.BlockSpec((1, D), lambda i: (0, 0)),
        ],
        out_specs=pl.BlockSpec((1, 2 * D), lambda i: (0, 0)),
        out_shape=jax.ShapeDtypeStruct((1, 2 * D), jnp.float32),
        scratch_shapes=[pltpu.VMEM((1, D), jnp.float32)],
    )(z, st, alpha, gamma, beta, r1)


@jax.jit
def _run(features, src, dst, edge_weights, W1, W2,
         gn1_alpha, gn1_gamma, gn1_beta, gn2_alpha, gn2_gamma, gn2_beta):
    ew16 = jnp.broadcast_to(edge_weights[:, None], (E, 16))
    degs = _deg_call(src, dst)[:, :, :DW]
    h0 = _scale_call(features, degs)
    agg1 = _spmm_call(h0, src, dst, ew16)
    z1, st1 = _mm_call(agg1, degs, W1)
    h1s, r1 = _gn_scale_call(z1, st1, degs, gn1_alpha, gn1_gamma, gn1_beta)
    agg2 = _spmm_call(h1s, src, dst, ew16)
    z2, st2 = _mm_call(agg2, degs, W2)
    return _final_call(z2, st2, gn2_alpha, gn2_gamma, gn2_beta, r1)


def kernel(features, edge_index, edge_weights, W1, W2,
           gn1_alpha, gn1_gamma, gn1_beta,
           gn2_alpha, gn2_gamma, gn2_beta):
    edge_index = edge_index.astype(jnp.int32)
    src = edge_index[0]
    dst = edge_index[1]
    return _run(features, src, dst, edge_weights, W1, W2,
                gn1_alpha.reshape(1, D), gn1_gamma.reshape(1, D),
                gn1_beta.reshape(1, D), gn2_alpha.reshape(1, D),
                gn2_gamma.reshape(1, D), gn2_beta.reshape(1, D))


# speedup vs baseline: 4.5115x; 1.1811x over previous
import functools
import jax
import jax.numpy as jnp
from jax import lax
from jax.experimental import pallas as pl
from jax.experimental.pallas import tpu as pltpu
from jax.experimental.pallas import tpu_sc as plsc

N = 10000
E = 160000
D = 256
HALF = 128
EPS = 1e-5
SLOPE = 0.01

NC = 2
NS = 16

E_PER_T = E // NS
CHUNK = 40
N_CHUNKS = E_PER_T // CHUNK
CP_ROWS = 624
TAIL = N - NS * CP_ROWS
ZR = 24

DCHUNK = CHUNK
DN_CHUNKS = E_PER_T // DCHUNK
DW = 16

_mesh = plsc.VectorSubcoreMesh(core_axis_name="c", subcore_axis_name="s")


def _leaky(x):
    return jnp.where(x >= 0, x, SLOPE * x)


DNB = 4


def _deg_body(src_hbm, dst_hbm, out_hbm,
              i0_v, i1_v, i2_v, i3_v, ones_v, deg_sh,
              si0, si1, si2, si3, ss0, ss1, ss2, ss3):
    cid = lax.axis_index("c")
    sid = lax.axis_index("s")
    ibuf = (i0_v, i1_v, i2_v, i3_v)
    sem_i = (si0, si1, si2, si3)
    sem_s = (ss0, ss1, ss2, ss3)
    base = sid * E_PER_T

    def zero_rows(r, _):
        for j in range(HALF // 16):
            ones_v[r, pl.ds(j * 16, 16)] = jnp.zeros((16,), jnp.float32)
        return 0

    def fill_ones(r, _):
        for j in range(HALF // 16):
            ones_v[r, pl.ds(j * 16, 16)] = jnp.ones((16,), jnp.float32)
        return 0

    lax.fori_loop(0, ZR, zero_rows, 0)
    for i in range(CP_ROWS // ZR):
        pltpu.sync_copy(ones_v.at[pl.ds(0, ZR)],
                        deg_sh.at[pl.ds(sid * CP_ROWS + i * ZR, ZR)])

    @pl.when(sid == 0)
    def _():
        pltpu.sync_copy(ones_v.at[pl.ds(0, TAIL)],
                        deg_sh.at[pl.ds(NS * CP_ROWS, TAIL)])

    lax.fori_loop(0, DCHUNK, fill_ones, 0)

    def _start_idx(b, k):
        off = base + k * DCHUNK

        @pl.when(cid == 0)
        def _():
            pltpu.async_copy(src_hbm.at[pl.ds(off, DCHUNK)], ibuf[b],
                             sem_i[b])

        @pl.when(cid == 1)
        def _():
            pltpu.async_copy(dst_hbm.at[pl.ds(off, DCHUNK)], ibuf[b],
                             sem_i[b])

    def _wait_idx(b, k):
        off = base + k * DCHUNK

        @pl.when(cid == 0)
        def _():
            pltpu.make_async_copy(src_hbm.at[pl.ds(off, DCHUNK)], ibuf[b],
                                  sem_i[b]).wait()

        @pl.when(cid == 1)
        def _():
            pltpu.make_async_copy(dst_hbm.at[pl.ds(off, DCHUNK)], ibuf[b],
                                  sem_i[b]).wait()

    def _wait_scat(b):
        pltpu.make_async_copy(ones_v, deg_sh.at[ibuf[b]], sem_s[b]).wait()

    _start_idx(0, 0)
    _start_idx(1, 1)
    plsc.subcore_barrier()

    def step(k, b):
        _wait_idx(b, k)
        pltpu.async_copy(ones_v, deg_sh.at[ibuf[b]], sem_s[b], add=True)
        bj = (b + 2) % DNB

        @pl.when(k >= 2)
        def _():
            _wait_scat(bj)

        @pl.when(k + 2 < DN_CHUNKS)
        def _():
            _start_idx(bj, k + 2)

    def chunk(k, _):
        for b in range(DNB):
            @pl.when(k % DNB == b)
            def _(b=b):
                step(k, b)
        return 0

    lax.fori_loop(0, DN_CHUNKS, chunk, 0)
    _wait_scat((DN_CHUNKS - 2) % DNB)
    _wait_scat((DN_CHUNKS - 1) % DNB)
    plsc.subcore_barrier()
    pltpu.sync_copy(deg_sh.at[pl.ds(sid * CP_ROWS, CP_ROWS)],
                    out_hbm.at[cid, pl.ds(sid * CP_ROWS, CP_ROWS)])

    @pl.when(sid == 0)
    def _():
        pltpu.sync_copy(deg_sh.at[pl.ds(NS * CP_ROWS, TAIL)],
                        out_hbm.at[cid, pl.ds(NS * CP_ROWS, TAIL)])


_deg_call = pl.kernel(
    _deg_body,
    out_type=jax.ShapeDtypeStruct((NC, N, HALF), jnp.float32),
    mesh=_mesh,
    scratch_types=[
        pltpu.VMEM((DCHUNK,), jnp.int32),
        pltpu.VMEM((DCHUNK,), jnp.int32),
        pltpu.VMEM((DCHUNK,), jnp.int32),
        pltpu.VMEM((DCHUNK,), jnp.int32),
        pltpu.VMEM((DCHUNK, HALF), jnp.float32),
        pltpu.VMEM_SHARED((N, HALF), jnp.float32),
        pltpu.SemaphoreType.DMA,
        pltpu.SemaphoreType.DMA,
        pltpu.SemaphoreType.DMA,
        pltpu.SemaphoreType.DMA,
        pltpu.SemaphoreType.DMA,
        pltpu.SemaphoreType.DMA,
        pltpu.SemaphoreType.DMA,
        pltpu.SemaphoreType.DMA,
    ],
)


def _spmm_body(h_hbm, src_hbm, dst_hbm, ew_hbm, out_hbm,
               sidx_v, didx_v, g0_v, g1_v, s0_v, s1_v, e0_v, e1_v, agg_sh,
               sem_g0, sem_g1, sem_e0, sem_e1, sem_s0, sem_s1):
    cid = lax.axis_index("c")
    sid = lax.axis_index("s")
    gbuf = (g0_v, g1_v)
    sbuf = (s0_v, s1_v)
    ebuf = (e0_v, e1_v)
    sem_g = (sem_g0, sem_g1)
    sem_e = (sem_e0, sem_e1)
    sem_s = (sem_s0, sem_s1)
    base = sid * E_PER_T

    def zero_rows(r, _):
        for j in range(HALF // 16):
            g0_v[r, pl.ds(j * 16, 16)] = jnp.zeros((16,), jnp.float32)
        return 0
    lax.fori_loop(0, ZR, zero_rows, 0)
    for i in range(CP_ROWS // ZR):
        pltpu.sync_copy(g0_v.at[pl.ds(0, ZR)],
                        agg_sh.at[pl.ds(sid * CP_ROWS + i * ZR, ZR)])

    @pl.when(sid == 0)
    def _():
        pltpu.sync_copy(g0_v.at[pl.ds(0, TAIL)],
                        agg_sh.at[pl.ds(NS * CP_ROWS, TAIL)])

    pltpu.sync_copy(src_hbm.at[pl.ds(base, E_PER_T)], sidx_v)
    pltpu.sync_copy(dst_hbm.at[pl.ds(base, E_PER_T)], didx_v)

    def _start_gather(b, k):
        pltpu.async_copy(
            h_hbm.at[cid].at[sidx_v.at[pl.ds(k * CHUNK, CHUNK)]],
            gbuf[b], sem_g[b])
        pltpu.async_copy(ew_hbm.at[pl.ds(base + k * CHUNK, CHUNK)],
                         ebuf[b], sem_e[b])

    def _wait_gather(b, k):
        pltpu.make_async_copy(
            h_hbm.at[cid].at[sidx_v.at[pl.ds(k * CHUNK, CHUNK)]],
            gbuf[b], sem_g[b]).wait()
        pltpu.make_async_copy(ew_hbm.at[pl.ds(base + k * CHUNK, CHUNK)],
                              ebuf[b], sem_e[b]).wait()

    def _start_scatter(b, k):
        pltpu.async_copy(
            sbuf[b], agg_sh.at[didx_v.at[pl.ds(k * CHUNK, CHUNK)]],
            sem_s[b], add=True)

    def _wait_scatter(b, k):
        pltpu.make_async_copy(
            sbuf[b], agg_sh.at[didx_v.at[pl.ds(k * CHUNK, CHUNK)]],
            sem_s[b]).wait()

    _start_gather(0, 0)
    _start_gather(1, 1)
    plsc.subcore_barrier()

    def step(k, b):
        _wait_gather(b, k)

        @pl.when(k >= 2)
        def _():
            _wait_scatter(b, k - 2)

        def scale(e, _):
            w = ebuf[b][e, :]
            for j in range(HALF // 16):
                sl = pl.ds(j * 16, 16)
                sbuf[b][e, sl] = gbuf[b][e, sl] * w
            return 0
        lax.fori_loop(0, CHUNK, scale, 0)

        @pl.when(k + 2 < N_CHUNKS)
        def _():
            _start_gather(b, k + 2)

        _start_scatter(b, k)

    def chunk(k, _):
        @pl.when(k % 2 == 0)
        def _():
            step(k, 0)

        @pl.when(k % 2 == 1)
        def _():
            step(k, 1)
        return 0

    lax.fori_loop(0, N_CHUNKS, chunk, 0)
    _wait_scatter((N_CHUNKS - 2) % 2, N_CHUNKS - 2)
    _wait_scatter((N_CHUNKS - 1) % 2, N_CHUNKS - 1)
    plsc.subcore_barrier()
    pltpu.sync_copy(agg_sh.at[pl.ds(sid * CP_ROWS, CP_ROWS)],
                    out_hbm.at[cid, pl.ds(sid * CP_ROWS, CP_ROWS)])

    @pl.when(sid == 0)
    def _():
        pltpu.sync_copy(agg_sh.at[pl.ds(NS * CP_ROWS, TAIL)],
                        out_hbm.at[cid, pl.ds(NS * CP_ROWS, TAIL)])


_spmm_call = pl.kernel(
    _spmm_body,
    out_type=jax.ShapeDtypeStruct((NC, N, HALF), jnp.float32),
    mesh=_mesh,
    scratch_types=[
        pltpu.VMEM((E_PER_T,), jnp.int32),
        pltpu.VMEM((E_PER_T,), jnp.int32),
        pltpu.VMEM((CHUNK, HALF), jnp.float32),
        pltpu.VMEM((CHUNK, HALF), jnp.float32),
        pltpu.VMEM((CHUNK, HALF), jnp.float32),
        pltpu.VMEM((CHUNK, HALF), jnp.float32),
        pltpu.VMEM((CHUNK, 16), jnp.float32),
        pltpu.VMEM((CHUNK, 16), jnp.float32),
        pltpu.VMEM_SHARED((N, HALF), jnp.float32),
        pltpu.SemaphoreType.DMA,
        pltpu.SemaphoreType.DMA,
        pltpu.SemaphoreType.DMA,
        pltpu.SemaphoreType.DMA,
        pltpu.SemaphoreType.DMA,
        pltpu.SemaphoreType.DMA,
    ],
)


BLK = 1000
NBLK = N // BLK


def _norm_from(deg_block):
    return lax.rsqrt(jnp.maximum(deg_block, 1.0))


def _scale_body(x_ref, degs_ref, out_ref):
    ns = _norm_from(degs_ref[0, :, 0:1])
    x = x_ref[...]
    out_ref[0] = x[:, :HALF] * ns
    out_ref[1] = x[:, HALF:] * ns


def _scale_call(features, degs):
    return pl.pallas_call(
        _scale_body,
        grid=(NBLK,),
        in_specs=[
            pl.BlockSpec((BLK, D), lambda i: (i, 0)),
            pl.BlockSpec((1, BLK, DW), lambda i: (0, i, 0)),
        ],
        out_specs=pl.BlockSpec((NC, BLK, HALF), lambda i: (0, i, 0)),
        out_shape=jax.ShapeDtypeStruct((NC, N, HALF), jnp.float32),
    )(features, degs)


def _mm_body(agg_ref, degs_ref, w_ref, z_ref, st_ref):
    i = pl.program_id(0)
    nd = _norm_from(degs_ref[0, :, 0:1])
    a0 = agg_ref[0] * nd
    a1 = agg_ref[1] * nd
    z = (jnp.dot(a0, w_ref[:HALF, :], preferred_element_type=jnp.float32) +
         jnp.dot(a1, w_ref[HALF:, :], preferred_element_type=jnp.float32))
    z_ref[...] = z
    s1 = jnp.sum(z, axis=0, keepdims=True)
    s2 = jnp.sum(z * z, axis=0, keepdims=True)
    st = jnp.concatenate([s1, s2], axis=0)

    @pl.when(i == 0)
    def _():
        st_ref[...] = st

    @pl.when(i > 0)
    def _():
        st_ref[...] = st_ref[...] + st


def _mm_call(agg, degs, w):
    return pl.pallas_call(
        _mm_body,
        grid=(NBLK,),
        in_specs=[
            pl.BlockSpec((NC, BLK, HALF), lambda i: (0, i, 0)),
            pl.BlockSpec((1, BLK, DW), lambda i: (1, i, 0)),
            pl.BlockSpec((D, D), lambda i: (0, 0)),
        ],
        out_specs=[
            pl.BlockSpec((BLK, D), lambda i: (i, 0)),
            pl.BlockSpec((2, D), lambda i: (0, 0)),
        ],
        out_shape=[
            jax.ShapeDtypeStruct((N, D), jnp.float32),
            jax.ShapeDtypeStruct((2, D), jnp.float32),
        ],
    )(agg, degs, w)


def _gnorm(z, st_ref, alpha_ref, gamma_ref, beta_ref):
    alpha = alpha_ref[...]
    m = st_ref[0:1, :] * (1.0 / N)
    var = st_ref[1:2, :] * (1.0 / N) + (alpha * alpha - 2.0 * alpha) * m * m
    inv = lax.rsqrt(var + EPS)
    return _leaky(gamma_ref[...] * inv * (z - alpha * m) + beta_ref[...])


def _gn_scale_body(z_ref, st_ref, degs_ref, a_ref, g_ref, b_ref,
                   out_ref, r_ref):
    i = pl.program_id(0)
    h = _gnorm(z_ref[...], st_ref, a_ref, g_ref, b_ref)
    r = jnp.sum(h, axis=0, keepdims=True)

    @pl.when(i == 0)
    def _():
        r_ref[...] = r

    @pl.when(i > 0)
    def _():
        r_ref[...] = r_ref[...] + r

    ns = _norm_from(degs_ref[0, :, 0:1])
    hs = h * ns
    out_ref[0] = hs[:, :HALF]
    out_ref[1] = hs[:, HALF:]


def _gn_scale_call(z, st, degs, alpha, gamma, beta):
    return pl.pallas_call(
        _gn_scale_body,
        grid=(NBLK,),
        in_specs=[
            pl.BlockSpec((BLK, D), lambda i: (i, 0)),
            pl.BlockSpec((2, D), lambda i: (0, 0)),
            pl.BlockSpec((1, BLK, DW), lambda i: (0, i, 0)),
            pl.BlockSpec((1, D), lambda i: (0, 0)),
            pl.BlockSpec((1, D), lambda i: (0, 0)),
            pl.BlockSpec((1, D), lambda i: (0, 0)),
        ],
        out_specs=[
            pl.BlockSpec((NC, BLK, HALF), lambda i: (0, i, 0)),
            pl.BlockSpec((1, D), lambda i: (0, 0)),
        ],
        out_shape=[
            jax.ShapeDtypeStruct((NC, N, HALF), jnp.float32),
            jax.ShapeDtypeStruct((1, D), jnp.float32),
        ],
    )(z, st, degs, alpha, gamma, beta)


def _final_body(z_ref, st_ref, a_ref, g_ref, b_ref, r1_ref, out_ref, racc):
    i = pl.program_id(0)
    h = _gnorm(z_ref[...], st_ref, a_ref, g_ref, b_ref)
    r = jnp.sum(h, axis=0, keepdims=True)

    @pl.when(i == 0)
    def _():
        racc[...] = r

    @pl.when(i > 0)
    def _():
        racc[...] = racc[...] + r

    @pl.when(i == NBLK - 1)
    def _():
        out_ref[0:1, :D] = _leaky(r1_ref[...] * (1.0 / N))
        out_ref[0:1, D:] = _leaky(racc[...] * (1.0 / N))


def _final_call(z, st, alpha, gamma, beta, r1):
    return pl.pallas_call(
        _final_body,
        grid=(NBLK,),
        in_specs=[
            pl.BlockSpec((BLK, D), lambda i: (i, 0)),
            pl.BlockSpec((2, D), lambda i: (0, 0)),
            pl.BlockSpec((1, D), lambda i: (0, 0)),
            pl.BlockSpec((1, D), lambda i: (0, 0)),
            pl.BlockSpec((1, D), lambda i: (0, 0)),
            pl.BlockSpec((1, D), lambda i: (0, 0)),
        ],
        out_specs=pl.BlockSpec((1, 2 * D), lambda i: (0, 0)),
        out_shape=jax.ShapeDtypeStruct((1, 2 * D), jnp.float32),
        scratch_shapes=[pltpu.VMEM((1, D), jnp.float32)],
    )(z, st, alpha, gamma, beta, r1)


@jax.jit
def _run(features, src, dst, edge_weights, W1, W2,
         gn1_alpha, gn1_gamma, gn1_beta, gn2_alpha, gn2_gamma, gn2_beta):
    ew16 = jnp.broadcast_to(edge_weights[:, None], (E, 16))
    degs = _deg_call(src, dst)[:, :, :DW]
    h0 = _scale_call(features, degs)
    agg1 = _spmm_call(h0, src, dst, ew16)
    z1, st1 = _mm_call(agg1, degs, W1)
    h1s, r1 = _gn_scale_call(z1, st1, degs, gn1_alpha, gn1_gamma, gn1_beta)
    agg2 = _spmm_call(h1s, src, dst, ew16)
    z2, st2 = _mm_call(agg2, degs, W2)
    return _final_call(z2, st2, gn2_alpha, gn2_gamma, gn2_beta, r1)


def kernel(features, edge_index, edge_weights, W1, W2,
           gn1_alpha, gn1_gamma, gn1_beta,
           gn2_alpha, gn2_gamma, gn2_beta):
    edge_index = edge_index.astype(jnp.int32)
    src = edge_index[0]
    dst = edge_index[1]
    return _run(features, src, dst, edge_weights, W1, W2,
                gn1_alpha.reshape(1, D), gn1_gamma.reshape(1, D),
                gn1_beta.reshape(1, D), gn2_alpha.reshape(1, D),
                gn2_gamma.reshape(1, D), gn2_beta.reshape(1, D))
